# Initial kernel scaffold; baseline (speedup 1.0000x reference)
import jax
import jax.numpy as jnp
from jax.experimental import pallas as pl


def kernel(x, ldist, rdist, Wv, pf1, pf2):
    raise NotImplementedError("write your pallas kernel here")


# speedup vs baseline: 3.1066x; 3.1066x over previous
import functools

import jax
import jax.numpy as jnp
from jax.experimental import pallas as pl
from jax.experimental.pallas import tpu as pltpu


def _round_up(n, m):
    return ((n + m - 1) // m) * m


def _gather_body(xi_ref,
                 ci_ref,
                 wv_ref,
                 lr_ref,
                 out_ref,
                 *, tm, ws, d, unroll):
    i = pl.program_id(0)
    base = i * tm

    def chunk(c, carry):
        b = base + c * unroll
        t0 = c * unroll
        for u in range(unroll):
            w = wv_ref[xi_ref[b + u], 0]
            e = lr_ref[ci_ref[b + u], 0]
            out_ref[t0 + u, 0, 0:ws] = w
            out_ref[t0 + u, 0, ws:d] = e
        return carry

    jax.lax.fori_loop(0, tm // unroll, chunk, 0)


@jax.jit
def kernel(x, ldist, rdist, Wv, pf1, pf2):
    B, S = x.shape
    WL, WS = Wv.shape
    FL, FS = pf1.shape
    FL2, FS2 = pf2.shape
    LR = FS + FS2
    D = WS + LR
    N = B * S

    xi = jnp.clip(x.reshape(N).astype(jnp.int32), 0, WL - 1)
    li = jnp.clip(ldist.reshape(N).astype(jnp.int32), 0, FL - 1)
    ri = jnp.clip(rdist.reshape(N).astype(jnp.int32), 0, FL2 - 1)
    ci = li * FL2 + ri

    tm = min(1024, _round_up(N, 8))
    n_pad = _round_up(N, tm)
    pad = n_pad - N
    if pad:
        zero = jnp.zeros((pad,), jnp.int32)
        xi = jnp.concatenate([xi, zero])
        ci = jnp.concatenate([ci, zero])

    lr_tab = jnp.concatenate(
        [jnp.broadcast_to(pf1[:, None, :], (FL, FL2, FS)),
         jnp.broadcast_to(pf2[None, :, :], (FL, FL2, FS2))],
        axis=-1).reshape(FL * FL2, LR)

    wv3 = Wv.reshape(WL, 1, WS)
    lr3 = lr_tab.reshape(FL * FL2, 1, LR)

    vmem_bytes = (WL * WS * 4 + FL * FL2 * LR * 4 + 2 * tm * D * 4
                  + (1 << 20))
    out = pl.pallas_call(
        functools.partial(_gather_body, tm=tm, ws=WS, d=D, unroll=8),
        out_shape=jax.ShapeDtypeStruct((n_pad, 1, D), jnp.float32),
        grid_spec=pltpu.PrefetchScalarGridSpec(
            num_scalar_prefetch=2,
            grid=(n_pad // tm,),
            in_specs=[
                pl.BlockSpec((WL, 1, WS), lambda i, xi, ci: (0, 0, 0)),
                pl.BlockSpec((FL * FL2, 1, LR), lambda i, xi, ci: (0, 0, 0)),
            ],
            out_specs=pl.BlockSpec((tm, 1, D), lambda i, xi, ci: (i, 0, 0)),
        ),
        compiler_params=pltpu.CompilerParams(
            dimension_semantics=("parallel",),
            vmem_limit_bytes=min(vmem_bytes, 60 * 1024 * 1024),
        ),
    )(xi, ci, wv3, lr3)

    return out[:N, 0, :].reshape(B, S, D)[:, None, :, :]
